# baseline (speedup 1.0000x reference)
import jax
import jax.numpy as jnp
from jax.experimental import pallas as pl

_IOU_THRESHOLD = 0.5
_MAX_DET = 512
_LANES = 128
_ROWS = 160
_NEG_INF = -1e30


def _nms_body(x1_ref, y1_ref, x2_ref, y2_ref, area_ref, sc_ref, out_ref):
    lane = jax.lax.broadcasted_iota(jnp.int32, (1, _LANES), 1)
    row2d = jax.lax.broadcasted_iota(jnp.int32, (_ROWS, _LANES), 0)

    x1 = x1_ref[...]
    y1 = y1_ref[...]
    x2 = x2_ref[...]
    y2 = y2_ref[...]
    area = area_ref[...]

    def pick(colmax, colrow):
        m = jnp.max(colmax, axis=1, keepdims=True)
        key = jnp.where(colmax == m, colrow * _LANES + lane,
                        jnp.int32(_ROWS * _LANES))
        best = jnp.min(key, axis=1, keepdims=True)
        return m, best, m > (_NEG_INF / 2)

    def extract(best):
        r_b = best // _LANES
        c_b = best - r_b * _LANES
        onehot = jnp.logical_and(row2d == r_b, lane == c_b)

        def ext(plane):
            return jnp.max(jnp.where(onehot, plane, -1.0), axis=(0, 1),
                           keepdims=True)

        return ext(x1), ext(y1), ext(x2), ext(y2)

    def iou_all(b):
        bx1, by1, bx2, by2 = b
        ix1 = jnp.maximum(bx1, x1)
        iy1 = jnp.maximum(by1, y1)
        ix2 = jnp.minimum(bx2, x2)
        iy2 = jnp.minimum(by2, y2)
        inter = jnp.clip(ix2 - ix1, 0.0) * jnp.clip(iy2 - iy1, 0.0)
        area_a = (bx2 - bx1) * (by2 - by1)
        return inter / (area_a + area - inter + 1e-8)

    def iou_pair(a, b):
        ax1, ay1, ax2, ay2 = a
        bx1, by1, bx2, by2 = b
        ix1 = jnp.maximum(ax1, bx1)
        iy1 = jnp.maximum(ay1, by1)
        ix2 = jnp.minimum(ax2, bx2)
        iy2 = jnp.minimum(ay2, by2)
        inter = jnp.clip(ix2 - ix1, 0.0) * jnp.clip(iy2 - iy1, 0.0)
        aa = (ax2 - ax1) * (ay2 - ay1)
        ab = (bx2 - bx1) * (by2 - by1)
        return inter / (aa + ab - inter + 1e-8)

    def row_of(m, b, valid):
        bx1, by1, bx2, by2 = b
        vf = jnp.where(valid, jnp.float32(1.0), jnp.float32(0.0))
        return ((jnp.where(lane == 0, bx1, 0.0)
                 + jnp.where(lane == 1, by1, 0.0)
                 + jnp.where(lane == 2, bx2, 0.0)
                 + jnp.where(lane == 3, by2, 0.0)
                 + jnp.where(lane == 4, m, 0.0)) * vf)[:, :5]

    def cond(state):
        count, _ = state
        return count < _MAX_DET

    def sweep(state):
        count, ms = state
        colmax = jnp.max(ms, axis=0, keepdims=True)
        colrow = jnp.min(jnp.where(ms == colmax, row2d, jnp.int32(_ROWS)),
                         axis=0, keepdims=True)
        m1, best1, valid1 = pick(colmax, colrow)
        b1 = extract(best1)
        iou1 = iou_all(b1)
        c1lane = best1 - (best1 // _LANES) * _LANES
        c1row = best1 // _LANES

        ms_m1 = jnp.where(jnp.logical_and(row2d == c1row, lane == c1lane),
                          jnp.float32(_NEG_INF), ms)
        v2col = jnp.max(ms_m1, axis=0, keepdims=True)
        r2col = jnp.min(jnp.where(ms_m1 == v2col, row2d, jnp.int32(_ROWS)),
                        axis=0, keepdims=True)
        colmax2 = jnp.where(lane == c1lane, v2col, colmax)
        colrow2 = jnp.where(lane == c1lane, r2col, colrow)
        m2, best2, valid2 = pick(colmax2, colrow2)
        b2 = extract(best2)
        iou2 = iou_all(b2)

        pair = iou_pair(b1, b2)
        accept2v = jnp.logical_and(valid2, pair <= _IOU_THRESHOLD)

        sup = jnp.logical_and(iou1 > _IOU_THRESHOLD, valid1)
        sup2 = jnp.logical_and(jnp.logical_and(iou2 > _IOU_THRESHOLD,
                                               accept2v), valid2)
        ms_new = jnp.where(jnp.logical_or(sup, sup2),
                           jnp.float32(_NEG_INF), ms)

        out_ref[pl.ds(count, 1), :] = row_of(m1, b1, valid1)
        accept2 = jnp.logical_and(accept2v.astype(jnp.int32)[0, 0] > 0,
                                  count < _MAX_DET - 1)

        @pl.when(accept2)
        def _():
            out_ref[pl.ds(count + 1, 1), :] = row_of(m2, b2, valid2)

        count_new = count + 1 + jnp.where(accept2, 1, 0)
        return count_new, ms_new

    jax.lax.while_loop(cond, sweep, (jnp.int32(0), sc_ref[...]))


def kernel(boxes, scores):
    n = boxes.shape[0]
    padded = _ROWS * _LANES
    pad = padded - n

    x1 = jnp.pad(boxes[:, 0], (0, pad)).reshape(_ROWS, _LANES)
    y1 = jnp.pad(boxes[:, 1], (0, pad)).reshape(_ROWS, _LANES)
    x2 = jnp.pad(boxes[:, 2], (0, pad)).reshape(_ROWS, _LANES)
    y2 = jnp.pad(boxes[:, 3], (0, pad)).reshape(_ROWS, _LANES)
    area = jnp.pad((boxes[:, 2] - boxes[:, 0]) * (boxes[:, 3] - boxes[:, 1]),
                   (0, pad)).reshape(_ROWS, _LANES)
    sc = jnp.pad(scores, (0, pad), constant_values=_NEG_INF).reshape(_ROWS, _LANES)

    return pl.pallas_call(
        _nms_body,
        out_shape=jax.ShapeDtypeStruct((_MAX_DET, 5), jnp.float32),
    )(x1, y1, x2, y2, area, sc)


# speedup vs baseline: 1.0333x; 1.0333x over previous
import jax
import jax.numpy as jnp
from jax.experimental import pallas as pl

_IOU_THRESHOLD = 0.5
_MAX_DET = 512
_LANES = 128
_ROWS = 160
_NEG_INF = -1e30


def _nms_body(x1_ref, y1_ref, x2_ref, y2_ref, area_ref, sc_ref, out_ref):
    lane = jax.lax.broadcasted_iota(jnp.int32, (1, _LANES), 1)
    row2d = jax.lax.broadcasted_iota(jnp.int32, (_ROWS, _LANES), 0)

    x1 = x1_ref[...]
    y1 = y1_ref[...]
    x2 = x2_ref[...]
    y2 = y2_ref[...]
    area = area_ref[...]

    def pick(colmax, colrow):
        m = jnp.max(colmax, axis=1, keepdims=True)
        key = jnp.where(colmax == m, colrow * _LANES + lane,
                        jnp.int32(_ROWS * _LANES))
        best = jnp.min(key, axis=1, keepdims=True)
        return m, best, m > (_NEG_INF / 2)

    def extract(bidx):
        r_b = bidx // _LANES
        c_b = bidx - r_b * _LANES
        onehot = (lane == c_b).astype(jnp.float32)

        def ext(ref):
            return jnp.sum(ref[pl.ds(r_b, 1), :] * onehot, axis=1,
                           keepdims=True)

        return ext(x1_ref), ext(y1_ref), ext(x2_ref), ext(y2_ref)

    def iou_all(b):
        bx1, by1, bx2, by2 = b
        ix1 = jnp.maximum(bx1, x1)
        iy1 = jnp.maximum(by1, y1)
        ix2 = jnp.minimum(bx2, x2)
        iy2 = jnp.minimum(by2, y2)
        inter = jnp.clip(ix2 - ix1, 0.0) * jnp.clip(iy2 - iy1, 0.0)
        area_a = (bx2 - bx1) * (by2 - by1)
        return inter / (area_a + area - inter + 1e-8)

    def iou_pair(a, b):
        ax1, ay1, ax2, ay2 = a
        bx1, by1, bx2, by2 = b
        ix1 = jnp.maximum(ax1, bx1)
        iy1 = jnp.maximum(ay1, by1)
        ix2 = jnp.minimum(ax2, bx2)
        iy2 = jnp.minimum(ay2, by2)
        inter = jnp.clip(ix2 - ix1, 0.0) * jnp.clip(iy2 - iy1, 0.0)
        aa = (ax2 - ax1) * (ay2 - ay1)
        ab = (bx2 - bx1) * (by2 - by1)
        return inter / (aa + ab - inter + 1e-8)

    def row_of(m, b, valid):
        bx1, by1, bx2, by2 = b
        vf = jnp.where(valid, jnp.float32(1.0), jnp.float32(0.0))
        return ((jnp.where(lane == 0, bx1, 0.0)
                 + jnp.where(lane == 1, by1, 0.0)
                 + jnp.where(lane == 2, bx2, 0.0)
                 + jnp.where(lane == 3, by2, 0.0)
                 + jnp.where(lane == 4, m, 0.0)) * vf)[:, :5]

    def cond(state):
        count, _ = state
        return count < _MAX_DET

    def sweep(state):
        count, ms = state
        colmax = jnp.max(ms, axis=0, keepdims=True)
        colrow = jnp.min(jnp.where(ms == colmax, row2d, jnp.int32(_ROWS)),
                         axis=0, keepdims=True)
        m1, best1, valid1 = pick(colmax, colrow)
        b1 = extract(best1[0, 0])
        iou1 = iou_all(b1)
        c1lane = best1 - (best1 // _LANES) * _LANES
        c1row = best1 // _LANES

        ms_m1 = jnp.where(jnp.logical_and(row2d == c1row, lane == c1lane),
                          jnp.float32(_NEG_INF), ms)
        v2col = jnp.max(ms_m1, axis=0, keepdims=True)
        r2col = jnp.min(jnp.where(ms_m1 == v2col, row2d, jnp.int32(_ROWS)),
                        axis=0, keepdims=True)
        colmax2 = jnp.where(lane == c1lane, v2col, colmax)
        colrow2 = jnp.where(lane == c1lane, r2col, colrow)
        m2, best2, valid2 = pick(colmax2, colrow2)
        b2 = extract(best2[0, 0])
        iou2 = iou_all(b2)

        pair = iou_pair(b1, b2)
        accept2v = jnp.logical_and(valid2, pair <= _IOU_THRESHOLD)

        sup = jnp.logical_and(iou1 > _IOU_THRESHOLD, valid1)
        sup2 = jnp.logical_and(jnp.logical_and(iou2 > _IOU_THRESHOLD,
                                               accept2v), valid2)
        ms_new = jnp.where(jnp.logical_or(sup, sup2),
                           jnp.float32(_NEG_INF), ms)

        out_ref[pl.ds(count, 1), :] = row_of(m1, b1, valid1)
        accept2 = jnp.logical_and(accept2v.astype(jnp.int32)[0, 0] > 0,
                                  count < _MAX_DET - 1)

        @pl.when(accept2)
        def _():
            out_ref[pl.ds(count + 1, 1), :] = row_of(m2, b2, valid2)

        count_new = count + 1 + jnp.where(accept2, 1, 0)
        return count_new, ms_new

    jax.lax.while_loop(cond, sweep, (jnp.int32(0), sc_ref[...]))


def kernel(boxes, scores):
    n = boxes.shape[0]
    padded = _ROWS * _LANES
    pad = padded - n

    x1 = jnp.pad(boxes[:, 0], (0, pad)).reshape(_ROWS, _LANES)
    y1 = jnp.pad(boxes[:, 1], (0, pad)).reshape(_ROWS, _LANES)
    x2 = jnp.pad(boxes[:, 2], (0, pad)).reshape(_ROWS, _LANES)
    y2 = jnp.pad(boxes[:, 3], (0, pad)).reshape(_ROWS, _LANES)
    area = jnp.pad((boxes[:, 2] - boxes[:, 0]) * (boxes[:, 3] - boxes[:, 1]),
                   (0, pad)).reshape(_ROWS, _LANES)
    sc = jnp.pad(scores, (0, pad), constant_values=_NEG_INF).reshape(_ROWS, _LANES)

    return pl.pallas_call(
        _nms_body,
        out_shape=jax.ShapeDtypeStruct((_MAX_DET, 5), jnp.float32),
    )(x1, y1, x2, y2, area, sc)
